# baseline (speedup 1.0000x reference)
import functools

import jax
import jax.numpy as jnp
from jax import lax
from jax.experimental import pallas as pl
from jax.experimental.pallas import tpu as pltpu
from jax.experimental.pallas import tpu_sc as plsc

B = 16384
V = 1000000
D = 16
NOH = 26
L = 20
NC = 13
F = 48
NW = 32
IDX_COLS = 128
XD = 448
CS = 16
GN = CS * F
G = GN // IDX_COLS
SAMPLES_PER_W = B // NW
N_GROUPS = SAMPLES_PER_W // CS
TOTAL_GROUPS = B // CS


def _vtake(vec, i):
    ids = jnp.full((16, 1), i, jnp.int32)
    dn = lax.GatherDimensionNumbers(offset_dims=(), collapsed_slice_dims=(0,),
                                    start_index_map=(0,))
    return lax.gather(vec, ids, dn, (1,),
                      mode=lax.GatherScatterMode.PROMISE_IN_BOUNDS)


def _sc_embed(meta, deep_table, wide16):
    mesh = plsc.VectorSubcoreMesh(core_axis_name="c", subcore_axis_name="s")

    @functools.partial(
        pl.kernel,
        out_type=(
            jax.ShapeDtypeStruct((B, XD), jnp.float32),
            jax.ShapeDtypeStruct((B, F), jnp.float32),
        ),
        mesh=mesh,
        scratch_types=[
            [pltpu.VMEM((4, G, IDX_COLS), jnp.int32) for _ in range(2)],
            [pltpu.VMEM((GN, D), jnp.float32) for _ in range(2)],
            [pltpu.VMEM((GN, D), jnp.float32) for _ in range(2)],
            [pltpu.VMEM((CS, XD), jnp.float32) for _ in range(2)],
            [pltpu.VMEM((CS, F), jnp.float32) for _ in range(2)],
            [pltpu.SemaphoreType.DMA for _ in range(2)],
            [pltpu.SemaphoreType.DMA for _ in range(2)],
        ],
        compiler_params=pltpu.CompilerParams(use_tc_tiling_on_sc=False,
                                             needs_layout_passes=False),
    )
    def k(meta_hbm, deep_hbm, wide_hbm, x_hbm, wv_hbm,
          meta_v, deep_v, wrow_v, x_v, wv_v, sem_d, sem_w):
        wid = lax.axis_index("s") * 2 + lax.axis_index("c")
        g0 = wid * N_GROUPS

        zero16 = jnp.zeros((16,), jnp.float32)
        for b in range(2):
            for ss in range(CS):
                x_v[b][ss, pl.ds(XD - 16, 16)] = zero16

        def fetch(gid, b):
            pltpu.sync_copy(meta_hbm.at[gid], meta_v[b])
            for j in range(G):
                pltpu.async_copy(
                    deep_hbm.at[meta_v[b].at[0, j]],
                    deep_v[b].at[pl.ds(j * IDX_COLS, IDX_COLS)], sem_d[b])
                pltpu.async_copy(
                    wide_hbm.at[meta_v[b].at[1, j]],
                    wrow_v[b].at[pl.ds(j * IDX_COLS, IDX_COLS)], sem_w[b])

        def wait_fetch(b):
            for j in range(G):
                pltpu.make_async_copy(
                    deep_hbm.at[meta_v[b].at[0, j]],
                    deep_v[b].at[pl.ds(j * IDX_COLS, IDX_COLS)],
                    sem_d[b]).wait()
                pltpu.make_async_copy(
                    wide_hbm.at[meta_v[b].at[1, j]],
                    wrow_v[b].at[pl.ds(j * IDX_COLS, IDX_COLS)],
                    sem_w[b]).wait()

        def compute(gid, b):
            mv, dv, wrv, xv, wvv = (meta_v[b], deep_v[b], wrow_v[b],
                                    x_v[b], wv_v[b])
            iota = lax.broadcasted_iota(jnp.int32, (16,), 0)

            def sample(ss, _):
                base = ss * F
                w = []
                for t in range(3):
                    p = base + 16 * t
                    w.append(plsc.bitcast(
                        mv[3, p >> 7, pl.ds(p & 127, 16)], jnp.float32))
                for f in range(NOH):
                    wb = _vtake(w[f // 16], f % 16)
                    xv[ss, pl.ds(16 * f, 16)] = dv[base + f, :] * wb
                acc = jnp.zeros((16,), jnp.float32)
                for f in range(NOH, NOH + L):
                    wb = _vtake(w[f // 16], f % 16)
                    acc = acc + dv[base + f, :] * wb
                xv[ss, pl.ds(16 * NOH, 16)] = acc
                for t in range(3):
                    p = base + 16 * t
                    lanev = mv[2, p >> 7, pl.ds(p & 127, 16)]
                    wp = plsc.load_gather(wrv, [iota + p, lanev]) * w[t]
                    wvv[ss, pl.ds(16 * t, 16)] = wp
                return 0

            lax.fori_loop(0, CS, sample, 0)
            pltpu.sync_copy(xv, x_hbm.at[pl.ds(gid * CS, CS)])
            pltpu.sync_copy(wvv, wv_hbm.at[pl.ds(gid * CS, CS)])

        fetch(g0, 0)

        def body(gg, _):
            for b in range(2):
                g = gg * 2 + b
                nb = (b + 1) % 2

                @pl.when(g + 1 < N_GROUPS)
                def _():
                    fetch(g0 + g + 1, nb)

                wait_fetch(b)
                compute(g0 + g, b)
            return 0

        lax.fori_loop(0, N_GROUPS // 2, body, 0)

    return k(meta, deep_table, wide16)


def _leaky(v):
    return jnp.where(v >= 0, v, 0.01 * v)


def _dot(a, b, dims):
    return jax.lax.dot_general(a, b, (dims, ((), ())),
                               preferred_element_type=jnp.float32,
                               precision=jax.lax.Precision.HIGHEST)


def _mlp_body(x_ref, wv_ref, ctns_ref, w2p_ref, w2c_ref, b2_ref, w3_ref,
              b3_ref, w4t_ref, b4_ref, out_ref):
    h = _dot(x_ref[...], w2p_ref[...], ((1,), (1,)))
    h += _dot(ctns_ref[...], w2c_ref[...], ((1,), (1,)))
    h = _leaky(h + b2_ref[...])
    h = _leaky(_dot(h, w3_ref[...], ((1,), (1,))) + b3_ref[...])
    h = _dot(h, w4t_ref[...], ((1,), (0,)))
    wide = 16.0 * jnp.sum(wv_ref[...], axis=1, keepdims=True)
    out_ref[...] = jax.nn.sigmoid(h + b4_ref[0, 0] + wide)


def _tc_mlp(x, wv, ctns, w2p, w2c, b2, w3, b3, w4t, b4):
    bt = 1024
    full = lambda shape: pl.BlockSpec(shape, lambda i: (0, 0))
    return pl.pallas_call(
        _mlp_body,
        grid=(B // bt,),
        in_specs=[
            pl.BlockSpec((bt, XD), lambda i: (i, 0)),
            pl.BlockSpec((bt, F), lambda i: (i, 0)),
            pl.BlockSpec((bt, NC), lambda i: (i, 0)),
            full((256, XD)),
            full((256, NC)),
            full((1, 256)),
            full((128, 256)),
            full((1, 128)),
            full((128, 1)),
            pl.BlockSpec(memory_space=pltpu.SMEM),
        ],
        out_specs=pl.BlockSpec((bt, 1), lambda i: (i, 0)),
        out_shape=jax.ShapeDtypeStruct((B, 1), jnp.float32),
    )(x, wv, ctns, w2p, w2c, b2, w3, b3, w4t, b4)


def kernel(onehot_i, onehot_x, multihot_list, ctns, wide_table, deep_table,
           w2, b2, w3, b3, w4, b4):
    mh_i = multihot_list[0, 0]
    mh_x = multihot_list[0, 1].astype(jnp.float32)
    zeros_i = jnp.zeros((B, F - NOH - L), jnp.int32)
    zeros_x = jnp.zeros((B, F - NOH - L), jnp.float32)
    idx = jnp.concatenate([onehot_i, mh_i, zeros_i], axis=1)
    wgt = jnp.concatenate([onehot_x, mh_x, zeros_x], axis=1)
    idx2 = idx.reshape(TOTAL_GROUPS, GN)
    wbits = jax.lax.bitcast_convert_type(wgt, jnp.int32).reshape(
        TOTAL_GROUPS, GN)
    meta = jnp.stack([idx2, idx2 >> 4, idx2 & 15, wbits],
                     axis=1).reshape(TOTAL_GROUPS, 4, G, IDX_COLS)

    wide_lin = jnp.take(wide_table, jnp.arange(V, dtype=jnp.int32),
                        axis=0, mode="clip")
    wide16 = wide_lin.reshape(V // 16, D)

    x, wv = _sc_embed(meta, deep_table, wide16)

    w2p = jnp.concatenate(
        [w2[:, :NOH * D + D], jnp.zeros((256, 16), jnp.float32)], axis=1)
    w2c = w2[:, NOH * D + D:]

    out = _tc_mlp(x, wv, ctns, w2p, w2c, b2.reshape(1, 256), w3,
                  b3.reshape(1, 128), w4.reshape(128, 1), b4.reshape(1, 1))
    return out.reshape(B)


# speedup vs baseline: 1.0413x; 1.0413x over previous
import functools

import jax
import jax.numpy as jnp
from jax import lax
from jax.experimental import pallas as pl
from jax.experimental.pallas import tpu as pltpu
from jax.experimental.pallas import tpu_sc as plsc

B = 16384
V = 1000000
D = 16
NOH = 26
L = 20
NC = 13
F = 48
NW = 32
IDX_COLS = 128
XD = 448
CS = 16
GN = CS * F
G = GN // IDX_COLS
SAMPLES_PER_W = B // NW
N_GROUPS = SAMPLES_PER_W // CS
TOTAL_GROUPS = B // CS


def _vtake(vec, i):
    ids = jnp.full((16, 1), i, jnp.int32)
    dn = lax.GatherDimensionNumbers(offset_dims=(), collapsed_slice_dims=(0,),
                                    start_index_map=(0,))
    return lax.gather(vec, ids, dn, (1,),
                      mode=lax.GatherScatterMode.PROMISE_IN_BOUNDS)


def _sc_embed(meta, deep_table, wide16):
    mesh = plsc.VectorSubcoreMesh(core_axis_name="c", subcore_axis_name="s")

    @functools.partial(
        pl.kernel,
        out_type=(
            jax.ShapeDtypeStruct((B, XD), jnp.float32),
            jax.ShapeDtypeStruct((B, F), jnp.float32),
        ),
        mesh=mesh,
        scratch_types=[
            [pltpu.VMEM((4, G, IDX_COLS), jnp.int32) for _ in range(2)],
            [pltpu.VMEM((GN, D), jnp.float32) for _ in range(2)],
            [pltpu.VMEM((GN, D), jnp.float32) for _ in range(2)],
            [pltpu.VMEM((CS, XD), jnp.float32) for _ in range(2)],
            [pltpu.VMEM((CS, F), jnp.float32) for _ in range(2)],
            [pltpu.SemaphoreType.DMA for _ in range(2)],
            [pltpu.SemaphoreType.DMA for _ in range(2)],
        ],
        compiler_params=pltpu.CompilerParams(use_tc_tiling_on_sc=False,
                                             needs_layout_passes=False),
    )
    def k(meta_hbm, deep_hbm, wide_hbm, x_hbm, wv_hbm,
          meta_v, deep_v, wrow_v, x_v, wv_v, sem_d, sem_w):
        wid = lax.axis_index("s") * 2 + lax.axis_index("c")
        g0 = wid * N_GROUPS

        zero16 = jnp.zeros((16,), jnp.float32)
        for b in range(2):
            for ss in range(CS):
                x_v[b][ss, pl.ds(XD - 16, 16)] = zero16

        def fetch(gid, b):
            pltpu.sync_copy(meta_hbm.at[gid], meta_v[b])
            for j in range(G):
                pltpu.async_copy(
                    deep_hbm.at[meta_v[b].at[0, j]],
                    deep_v[b].at[pl.ds(j * IDX_COLS, IDX_COLS)], sem_d[b])
                pltpu.async_copy(
                    wide_hbm.at[meta_v[b].at[1, j]],
                    wrow_v[b].at[pl.ds(j * IDX_COLS, IDX_COLS)], sem_w[b])

        def wait_fetch(b):
            for j in range(G):
                pltpu.make_async_copy(
                    deep_hbm.at[meta_v[b].at[0, j]],
                    deep_v[b].at[pl.ds(j * IDX_COLS, IDX_COLS)],
                    sem_d[b]).wait()
                pltpu.make_async_copy(
                    wide_hbm.at[meta_v[b].at[1, j]],
                    wrow_v[b].at[pl.ds(j * IDX_COLS, IDX_COLS)],
                    sem_w[b]).wait()

        def compute(gid, b):
            mv, dv, wrv, xv, wvv = (meta_v[b], deep_v[b], wrow_v[b],
                                    x_v[b], wv_v[b])
            iota = lax.broadcasted_iota(jnp.int32, (16,), 0)

            def sample(ss, _):
                base = ss * F
                w = []
                for t in range(3):
                    p = base + 16 * t
                    w.append(plsc.bitcast(
                        mv[3, p >> 7, pl.ds(p & 127, 16)], jnp.float32))
                for f in range(NOH):
                    wb = _vtake(w[f // 16], f % 16)
                    xv[ss, pl.ds(16 * f, 16)] = dv[base + f, :] * wb
                acc = jnp.zeros((16,), jnp.float32)
                for f in range(NOH, NOH + L):
                    wb = _vtake(w[f // 16], f % 16)
                    acc = acc + dv[base + f, :] * wb
                xv[ss, pl.ds(16 * NOH, 16)] = acc
                for t in range(3):
                    p = base + 16 * t
                    lanev = mv[2, p >> 7, pl.ds(p & 127, 16)]
                    wp = plsc.load_gather(wrv, [iota + p, lanev]) * w[t]
                    wvv[ss, pl.ds(16 * t, 16)] = wp
                return 0

            lax.fori_loop(0, CS, sample, 0)
            pltpu.sync_copy(xv, x_hbm.at[pl.ds(gid * CS, CS)])
            pltpu.sync_copy(wvv, wv_hbm.at[pl.ds(gid * CS, CS)])

        fetch(g0, 0)

        def body(gg, _):
            for b in range(2):
                g = gg * 2 + b
                nb = (b + 1) % 2

                @pl.when(g + 1 < N_GROUPS)
                def _():
                    fetch(g0 + g + 1, nb)

                wait_fetch(b)
                compute(g0 + g, b)
            return 0

        lax.fori_loop(0, N_GROUPS // 2, body, 0)

    return k(meta, deep_table, wide16)


def _leaky(v):
    return jnp.where(v >= 0, v, 0.01 * v)


def _dot(a, b, dims):
    return jax.lax.dot_general(a, b, (dims, ((), ())),
                               preferred_element_type=jnp.float32,
                               precision=jax.lax.Precision.HIGHEST)


def _mlp_body(x_ref, wv_ref, ctns_ref, w2p_ref, w2c_ref, b2_ref, w3_ref,
              b3_ref, w4t_ref, b4_ref, out_ref):
    h = _dot(x_ref[...], w2p_ref[...], ((1,), (1,)))
    h += _dot(ctns_ref[...], w2c_ref[...], ((1,), (1,)))
    h = _leaky(h + b2_ref[...])
    h = _leaky(_dot(h, w3_ref[...], ((1,), (1,))) + b3_ref[...])
    h = _dot(h, w4t_ref[...], ((1,), (0,)))
    wide = 16.0 * jnp.sum(wv_ref[...], axis=1, keepdims=True)
    out_ref[...] = jax.nn.sigmoid(h + b4_ref[0, 0] + wide)


def _tc_mlp(x, wv, ctns, w2p, w2c, b2, w3, b3, w4t, b4):
    bt = 1024
    full = lambda shape: pl.BlockSpec(shape, lambda i: (0, 0))
    return pl.pallas_call(
        _mlp_body,
        grid=(B // bt,),
        in_specs=[
            pl.BlockSpec((bt, XD), lambda i: (i, 0)),
            pl.BlockSpec((bt, F), lambda i: (i, 0)),
            pl.BlockSpec((bt, NC), lambda i: (i, 0)),
            full((256, XD)),
            full((256, NC)),
            full((1, 256)),
            full((128, 256)),
            full((1, 128)),
            full((128, 1)),
            pl.BlockSpec(memory_space=pltpu.SMEM),
        ],
        out_specs=pl.BlockSpec((bt, 1), lambda i: (i, 0)),
        out_shape=jax.ShapeDtypeStruct((B, 1), jnp.float32),
    )(x, wv, ctns, w2p, w2c, b2, w3, b3, w4t, b4)


def kernel(onehot_i, onehot_x, multihot_list, ctns, wide_table, deep_table,
           w2, b2, w3, b3, w4, b4):
    mh_i = multihot_list[0, 0]
    mh_x = multihot_list[0, 1].astype(jnp.float32)
    zeros_i = jnp.zeros((B, F - NOH - L), jnp.int32)
    zeros_x = jnp.zeros((B, F - NOH - L), jnp.float32)
    idx = jnp.concatenate([onehot_i, mh_i, zeros_i], axis=1)
    wgt = jnp.concatenate([onehot_x, mh_x, zeros_x], axis=1)
    idx2 = idx.reshape(TOTAL_GROUPS, GN)
    wbits = jax.lax.bitcast_convert_type(wgt, jnp.int32).reshape(
        TOTAL_GROUPS, GN)
    meta = jnp.stack([idx2, idx2 >> 4, idx2 & 15, wbits],
                     axis=1).reshape(TOTAL_GROUPS, 4, G, IDX_COLS)

    x, wv = _sc_embed(meta, deep_table, wide_table.reshape(V // 16, D))

    w2p = jnp.concatenate(
        [w2[:, :NOH * D + D], jnp.zeros((256, 16), jnp.float32)], axis=1)
    w2c = w2[:, NOH * D + D:]

    out = _tc_mlp(x, wv, ctns, w2p, w2c, b2.reshape(1, 256), w3,
                  b3.reshape(1, 128), w4.reshape(128, 1), b4.reshape(1, 1))
    return out.reshape(B)
